# Initial kernel scaffold; baseline (speedup 1.0000x reference)
import jax
import jax.numpy as jnp
from jax.experimental import pallas as pl


def kernel(indices, table, traits):
    raise NotImplementedError("write your pallas kernel here")


# speedup vs baseline: 2.4372x; 2.4372x over previous
import functools

import jax
import jax.numpy as jnp
from jax import lax
from jax.experimental import pallas as pl
from jax.experimental.pallas import tpu as pltpu
from jax.experimental.pallas import tpu_sc as plsc


def _traits_gather_tc(indices2d, traits_flat, num_personas):
    r, c = indices2d.shape

    def body(idx_ref, tr_ref, out_ref):
        idx = idx_ref[...]
        for t in range(2):
            acc = jnp.zeros((r, c), jnp.float32)
            for v in range(num_personas):
                acc = jnp.where(idx == v, tr_ref[2 * v + t], acc)
            out_ref[t] = acc

    return pl.pallas_call(
        body,
        out_shape=jax.ShapeDtypeStruct((2, r, c), jnp.float32),
        in_specs=[
            pl.BlockSpec(memory_space=pltpu.VMEM),
            pl.BlockSpec(memory_space=pltpu.SMEM),
        ],
        out_specs=pl.BlockSpec(memory_space=pltpu.VMEM),
    )(indices2d, traits_flat)


def _make_lookup(B, V, D, T, num_cores, num_subcores):
    nw = num_cores * num_subcores
    b_per_w = B // nw
    n_chunks = 2
    chunk = b_per_w // n_chunks
    mesh = plsc.VectorSubcoreMesh(core_axis_name="c", subcore_axis_name="s")

    @functools.partial(
        pl.kernel,
        mesh=mesh,
        out_type=jax.ShapeDtypeStruct((B, D + T), jnp.float32),
        scratch_types=[
            pltpu.VMEM((b_per_w,), jnp.int32),
            pltpu.VMEM((chunk, D), jnp.float32),
            pltpu.VMEM((chunk, T), jnp.float32),
            pltpu.SemaphoreType.DMA,
        ],
    )
    def lookup(tbl_hbm, trp_hbm, idx_hbm, out_hbm, idx_v, emb_v, trv_v, sem):
        wid = lax.axis_index("s") * num_cores + lax.axis_index("c")
        base = wid * b_per_w
        pltpu.sync_copy(idx_hbm.at[pl.ds(base, b_per_w)], idx_v)
        for ch in range(n_chunks):
            off = ch * chunk
            idx_ch = idx_v.at[pl.ds(off, chunk)]
            cp_e = pltpu.async_copy(tbl_hbm.at[idx_ch], emb_v, sem)
            pltpu.sync_copy(trp_hbm.at[pl.ds(base + off, chunk)], trv_v)
            cp_e.wait()
            pltpu.sync_copy(emb_v, out_hbm.at[pl.ds(base + off, chunk), pl.ds(0, D)])
            pltpu.sync_copy(trv_v, out_hbm.at[pl.ds(base + off, chunk), pl.ds(D, T)])

    return lookup


def kernel(indices, table, traits):
    B = indices.shape[0]
    V, D = table.shape
    T = traits.shape[1]
    rows = B // 128
    tr3d = _traits_gather_tc(indices.reshape(rows, 128), traits.reshape(-1), V)
    tr_pairs = tr3d.reshape(2, B).T
    info = plsc.get_sparse_core_info()
    lookup = _make_lookup(B, V, D, T, info.num_cores, info.num_subcores)
    return lookup(table, tr_pairs, indices)
